# Initial kernel scaffold; baseline (speedup 1.0000x reference)
import jax
import jax.numpy as jnp
from jax.experimental import pallas as pl


def kernel(inputs, params):
    raise NotImplementedError("write your pallas kernel here")


# speedup vs baseline: 5.6718x; 5.6718x over previous
import functools

import jax
import jax.numpy as jnp
from jax.experimental import pallas as pl
from jax.experimental.pallas import tpu as pltpu

EPS = 1e-12
KG = 32
BIG = 1e30


def _mm(a, b):
    return jax.lax.dot_general(
        a, b, (((1,), (0,)), ((), ())),
        precision=jax.lax.Precision.HIGHEST,
        preferred_element_type=jnp.float32)


def _flat_iota(shape):
    r = jax.lax.broadcasted_iota(jnp.int32, shape, 0)
    c = jax.lax.broadcasted_iota(jnp.int32, shape, 1)
    return r * shape[1] + c


def _fps_level(X, Y, Z, n_valid, m, out_rows):
    iota_n = _flat_iota(X.shape)
    n_lanes = X.shape[0] * 128
    valid = iota_n < n_valid
    d0 = jnp.where(valid, jnp.float32(1e10), jnp.float32(-1.0))
    mshape = (out_rows, 128)
    iota_m = _flat_iota(mshape)
    idx0 = jnp.zeros(mshape, jnp.int32)
    z0 = jnp.zeros(mshape, jnp.float32)

    def body(i, carry):
        d, last, idxa, cxa, cya, cza = carry
        sel = iota_n == last
        xl = jnp.sum(jnp.where(sel, X, 0.0))
        yl = jnp.sum(jnp.where(sel, Y, 0.0))
        zl = jnp.sum(jnp.where(sel, Z, 0.0))
        dd = jnp.sqrt((X - xl) ** 2 + (Y - yl) ** 2 + (Z - zl) ** 2 + EPS)
        d = jnp.minimum(d, jnp.where(valid, dd, jnp.float32(-1.0)))
        put = iota_m == i
        idxa = jnp.where(put, last, idxa)
        cxa = jnp.where(put, xl, cxa)
        cya = jnp.where(put, yl, cya)
        cza = jnp.where(put, zl, cza)
        mx = jnp.max(d)
        nxt = jnp.min(jnp.where(d == mx, iota_n, n_lanes)).astype(jnp.int32)
        return d, nxt, idxa, cxa, cya, cza

    d, _, idxa, cxa, cya, cza = jax.lax.fori_loop(
        0, m, body, (d0, jnp.int32(0), idx0, z0, z0, z0))
    return idxa, cxa, cya, cza, jnp.max(d)


def _fps_kernel(xs_ref, ys_ref, zs_ref,
                idx0_ref, c0x_ref, c0y_ref, c0z_ref,
                idx1_ref, c1x_ref, c1y_ref, c1z_ref,
                idx2_ref, c2x_ref, c2y_ref, c2z_ref,
                scal_ref):
    X0, Y0, Z0 = xs_ref[...], ys_ref[...], zs_ref[...]
    idx0, c0x, c0y, c0z, maxd0 = _fps_level(X0, Y0, Z0, 4096, 512, 4)
    idx0_ref[...] = idx0
    c0x_ref[...] = c0x
    c0y_ref[...] = c0y
    c0z_ref[...] = c0z

    idx1, c1x, c1y, c1z, maxd1 = _fps_level(c0x, c0y, c0z, 512, 64, 1)
    idx1_ref[...] = idx1
    c1x_ref[...] = c1x
    c1y_ref[...] = c1y
    c1z_ref[...] = c1z

    idx2, c2x, c2y, c2z, maxd2 = _fps_level(c1x, c1y, c1z, 64, 16, 1)
    idx2_ref[...] = idx2
    c2x_ref[...] = c2x
    c2y_ref[...] = c2y
    c2z_ref[...] = c2z

    iota = _flat_iota((1, 128))
    m8 = iota < 16
    gx = jnp.sum(jnp.where(m8, c2x, 0.0)) / 16.0
    gy = jnp.sum(jnp.where(m8, c2y, 0.0)) / 16.0
    gz = jnp.sum(jnp.where(m8, c2z, 0.0)) / 16.0
    dc = jnp.sqrt((c2x - gx) ** 2 + (c2y - gy) ** 2 + (c2z - gz) ** 2 + EPS)
    dc = jnp.where(m8, dc, jnp.float32(BIG))
    mn = jnp.min(dc)
    cidx = jnp.min(jnp.where(dc == mn, iota, 128)).astype(jnp.int32)
    csel = iota == cidx
    ccx = jnp.sum(jnp.where(csel, c2x, 0.0))
    ccy = jnp.sum(jnp.where(csel, c2y, 0.0))
    ccz = jnp.sum(jnp.where(csel, c2z, 0.0))

    lane = iota
    scal = jnp.zeros((1, 128), jnp.float32)
    for pos_i, val in enumerate([maxd0, maxd1, maxd2, ccx, ccy, ccz]):
        scal = jnp.where(lane == pos_i, val, scal)
    scal_ref[...] = scal


def _topk_kernel(px_ref, py_ref, pz_ref, cx_ref, cy_ref, cz_ref,
                 cidx_ref, maxd_ref, nbrT_ref, *, n):
    px, py, pz = px_ref[...], py_ref[...], pz_ref[...]
    cx, cy, cz = cx_ref[...], cy_ref[...], cz_ref[...]
    D = jnp.sqrt((px - cx) ** 2 + (py - cy) ** 2 + (pz - cz) ** 2 + EPS)
    radius = 2.0 * maxd_ref[0]
    row_iota = jax.lax.broadcasted_iota(jnp.int32, D.shape, 0)
    cidx = cidx_ref[...]
    for t in range(KG):
        vmin = jnp.min(D, axis=0, keepdims=True)
        sel = jnp.min(jnp.where(D == vmin, row_iota, n),
                      axis=0, keepdims=True)
        nbrT_ref[t:t + 1, :] = jnp.where(vmin <= radius, sel, cidx)
        D = jnp.where(row_iota == sel, jnp.float32(BIG), D)


def _sa_kernel(nbr_ref, data_ref, cpad_ref,
               w1_ref, b1_ref, w2_ref, b2_ref, w3_ref, b3_ref,
               out_ref, scratch_ref, *, rows):
    def body(j, carry):
        idx = nbr_ref[j]
        scratch_ref[pl.ds(j, 1), :] = data_ref[pl.ds(idx, 1), :]
        return carry

    jax.lax.fori_loop(0, rows, body, 0, unroll=8)
    feat = scratch_ref[...] - cpad_ref[...]
    h = jnp.tanh(_mm(feat, w1_ref[...]) + b1_ref[...])
    h = jnp.tanh(_mm(h, w2_ref[...]) + b2_ref[...])
    h = jnp.tanh(_mm(h, w3_ref[...]) + b3_ref[...])
    m = rows // KG
    acc = h[0:m, :]
    for t in range(1, KG):
        acc = jnp.maximum(acc, h[t * m:(t + 1) * m, :])
    out_ref[...] = acc


def _global_kernel(data_ref, crow_ref, g1_ref, bg1_ref, g2_ref, bg2_ref,
                   g3_ref, bg3_ref, pospad_ref, d1a_ref, d1b_ref, bd1_ref,
                   d2_ref, bd2_ref, out_ref):
    f = data_ref[...] - crow_ref[...]
    h = jnp.tanh(_mm(f, g1_ref[...]) + bg1_ref[...])
    h = jnp.tanh(_mm(h, g2_ref[...]) + bg2_ref[...])
    h = jnp.tanh(_mm(h, g3_ref[...]) + bg3_ref[...])
    pooled = jnp.max(h, axis=0, keepdims=True)
    h2 = jnp.tanh(_mm(pospad_ref[...], d1a_ref[...])
                  + _mm(pooled, d1b_ref[...]) + bd1_ref[...])
    out_ref[...] = jnp.tanh(_mm(h2, d2_ref[...]) + bd2_ref[...])


def _interp_kernel(ux_ref, uy_ref, uz_ref, kx_ref, ky_ref, kz_ref,
                   cur_ref, pospad_ref, w1a_ref, w1b_ref, b1_ref,
                   w2_ref, b2_ref, out_ref, *, k_valid):
    ux, uy, uz = ux_ref[...], uy_ref[...], uz_ref[...]
    kx, ky, kz = kx_ref[...], ky_ref[...], kz_ref[...]
    D = jnp.sqrt((ux - kx) ** 2 + (uy - ky) ** 2 + (uz - kz) ** 2 + EPS)
    kpad = D.shape[1]
    col_iota = jax.lax.broadcasted_iota(jnp.int32, D.shape, 1)
    D = jnp.where(col_iota < k_valid, D, jnp.float32(BIG))
    vals, sels = [], []
    for _ in range(3):
        vmin = jnp.min(D, axis=1, keepdims=True)
        sel = jnp.min(jnp.where(D == vmin, col_iota, kpad),
                      axis=1, keepdims=True)
        vals.append(vmin)
        sels.append(sel)
        D = jnp.where(col_iota == sel, jnp.float32(BIG), D)
    ws = [1.0 / jnp.maximum(v, 1e-10) for v in vals]
    wsum = ws[0] + ws[1] + ws[2]
    Wmat = jnp.zeros_like(D)
    for t in range(3):
        Wmat = Wmat + jnp.where(col_iota == sels[t], ws[t] / wsum, 0.0)
    interp = _mm(Wmat, cur_ref[...])
    h = jnp.tanh(_mm(pospad_ref[...], w1a_ref[...])
                 + _mm(interp, w1b_ref[...]) + b1_ref[...])
    out_ref[...] = jnp.tanh(_mm(h, w2_ref[...]) + b2_ref[...])


def _pad_cols(a, width):
    return jnp.concatenate(
        [a, jnp.zeros((a.shape[0], width - a.shape[1]), a.dtype)], axis=1)


def _pad_rows(a, height):
    return jnp.concatenate(
        [a, jnp.zeros((height - a.shape[0], a.shape[1]), a.dtype)], axis=0)


def _f32(shape):
    return jax.ShapeDtypeStruct(shape, jnp.float32)


def _i32(shape):
    return jax.ShapeDtypeStruct(shape, jnp.int32)


def _run_topk(pcols, crow, cidxrow, maxd, n, mpad):
    fn = functools.partial(_topk_kernel, n=n)
    return pl.pallas_call(
        fn,
        in_specs=[pl.BlockSpec(memory_space=pltpu.VMEM)] * 6
        + [pl.BlockSpec(memory_space=pltpu.VMEM),
           pl.BlockSpec(memory_space=pltpu.SMEM)],
        out_shape=_i32((KG, mpad)),
    )(pcols[0], pcols[1], pcols[2], crow[0], crow[1], crow[2],
      cidxrow, maxd)


def _run_sa(nbr_flat, data, cpad, layers, rows, cout):
    w1, b1 = layers[0]
    w2, b2 = layers[1]
    w3, b3 = layers[2]
    cpadw = data.shape[1]
    w1p = _pad_rows(w1, cpadw)
    fn = functools.partial(_sa_kernel, rows=rows)
    return pl.pallas_call(
        fn,
        in_specs=[pl.BlockSpec(memory_space=pltpu.SMEM)]
        + [pl.BlockSpec(memory_space=pltpu.VMEM)] * 8,
        out_shape=_f32((rows // KG, cout)),
        scratch_shapes=[pltpu.VMEM((rows, cpadw), jnp.float32)],
    )(nbr_flat, data, cpad,
      w1p, b1.reshape(1, -1), w2, b2.reshape(1, -1), w3, b3.reshape(1, -1))


def _run_interp(upcols, krow, cur_pad, pospad, layers, k_valid, uk, cout):
    (w1, b1), (w2, b2) = layers
    w1a = _pad_rows(w1[:3], 128)
    w1b = w1[3:]
    fn = functools.partial(_interp_kernel, k_valid=k_valid)
    return pl.pallas_call(
        fn,
        out_shape=_f32((uk, cout)),
    )(upcols[0], upcols[1], upcols[2], krow[0], krow[1], krow[2],
      cur_pad, pospad, w1a, w1b, b1.reshape(1, -1), w2, b2.reshape(1, -1))


def kernel(inputs, params):
    pos = inputs[:, :3]
    xs = inputs[:, 0].reshape(32, 128)
    ys = inputs[:, 1].reshape(32, 128)
    zs = inputs[:, 2].reshape(32, 128)

    a_out = pl.pallas_call(
        _fps_kernel,
        out_shape=[_i32((4, 128)), _f32((4, 128)), _f32((4, 128)),
                   _f32((4, 128)),
                   _i32((1, 128)), _f32((1, 128)), _f32((1, 128)),
                   _f32((1, 128)),
                   _i32((1, 128)), _f32((1, 128)), _f32((1, 128)),
                   _f32((1, 128)),
                   _f32((1, 128))],
    )(xs, ys, zs)
    (idx0, c0x, c0y, c0z, idx1, c1x, c1y, c1z,
     idx2, c2x, c2y, c2z, scal) = a_out

    conv = params["conv"]
    conv_re = params["conv_re"]

    pcols0 = [pos[:, 0:1], pos[:, 1:2], pos[:, 2:3]]
    crow0 = [c0x.reshape(1, 512), c0y.reshape(1, 512), c0z.reshape(1, 512)]
    nbrT0 = _run_topk(pcols0, crow0, idx0.reshape(1, 512),
                      scal[0, 0:1], 4096, 512)
    cps0 = jnp.stack([c0x.reshape(512), c0y.reshape(512),
                      c0z.reshape(512)], axis=1)
    cpad0 = jnp.tile(_pad_cols(cps0, 64), (KG, 1))
    pooled0 = _run_sa(nbrT0.reshape(-1), inputs, cpad0,
                      conv[0], KG * 512, 128)

    data1 = _pad_cols(jnp.concatenate([cps0, pooled0], axis=1), 256)
    pcols1 = [cps0[:, 0:1], cps0[:, 1:2], cps0[:, 2:3]]
    crow1 = [c1x, c1y, c1z]
    nbrT1 = _run_topk(pcols1, crow1, idx1, scal[0, 1:2], 512, 128)
    cps1 = jnp.stack([c1x[0, :64], c1y[0, :64], c1z[0, :64]], axis=1)
    cpad1 = jnp.tile(_pad_cols(cps1, 256), (KG, 1))
    pooled1 = _run_sa(nbrT1[:, :64].reshape(-1), data1, cpad1,
                      conv[1], KG * 64, 256)

    data2 = _pad_cols(jnp.concatenate([cps1, pooled1], axis=1), 384)
    pcols2 = [cps1[:, 0:1], cps1[:, 1:2], cps1[:, 2:3]]
    crow2 = [c2x, c2y, c2z]
    nbrT2 = _run_topk(pcols2, crow2, idx2, scal[0, 2:3], 64, 128)
    cps2 = jnp.stack([c2x[0, :16], c2y[0, :16], c2z[0, :16]], axis=1)
    cpad2 = jnp.tile(_pad_cols(cps2, 384), (KG, 1))
    pooled2 = _run_sa(nbrT2[:, :16].reshape(-1), data2, cpad2,
                      conv[2], KG * 16, 512)

    data3 = _pad_cols(jnp.concatenate([cps2, pooled2], axis=1), 640)
    crow = _pad_cols(scal[0:1, 3:6], 640)
    (g1, bg1), (g2, bg2), (g3, bg3) = conv[3]
    (rd1, rb1), (rd2, rb2) = conv_re[0]
    cur3 = pl.pallas_call(
        _global_kernel,
        out_shape=_f32((16, 512)),
    )(data3, crow, _pad_rows(g1, 640), bg1.reshape(1, -1),
      g2, bg2.reshape(1, -1), g3, bg3.reshape(1, -1),
      _pad_cols(cps2, 128), _pad_rows(rd1[:3], 128), rd1[3:],
      rb1.reshape(1, -1), rd2, rb2.reshape(1, -1))

    cur2 = _run_interp(pcols2, crow2, _pad_rows(cur3, 128),
                       _pad_cols(cps1, 128), conv_re[1], 16, 64, 256)

    cur1 = _run_interp(pcols1, crow1, _pad_rows(cur2, 128),
                       _pad_cols(cps0, 128), conv_re[2], 64, 512, 128)

    out = _run_interp(pcols0, crow0, cur1,
                      _pad_cols(pos, 128), conv_re[3], 512, 4096, 128)
    return out


# speedup vs baseline: 5.7772x; 1.0186x over previous
import functools

import jax
import jax.numpy as jnp
from jax.experimental import pallas as pl
from jax.experimental.pallas import tpu as pltpu
from jax.experimental.pallas import tpu_sc as plsc

EPS = 1e-12
KG = 32
BIG = 1e30


def _mm(a, b):
    return jax.lax.dot_general(
        a, b, (((1,), (0,)), ((), ())),
        precision=jax.lax.Precision.HIGHEST,
        preferred_element_type=jnp.float32)


def _flat_iota(shape):
    r = jax.lax.broadcasted_iota(jnp.int32, shape, 0)
    c = jax.lax.broadcasted_iota(jnp.int32, shape, 1)
    return r * shape[1] + c


def _fps_level(X, Y, Z, n_valid, m, out_rows):
    iota_n = _flat_iota(X.shape)
    n_lanes = X.shape[0] * 128
    valid = iota_n < n_valid
    d0 = jnp.where(valid, jnp.float32(1e10), jnp.float32(-1.0))
    mshape = (out_rows, 128)
    iota_m = _flat_iota(mshape)
    idx0 = jnp.zeros(mshape, jnp.int32)
    z0 = jnp.zeros(mshape, jnp.float32)

    def body(i, carry):
        d, last, idxa, cxa, cya, cza = carry
        sel = iota_n == last
        xl = jnp.sum(jnp.where(sel, X, 0.0))
        yl = jnp.sum(jnp.where(sel, Y, 0.0))
        zl = jnp.sum(jnp.where(sel, Z, 0.0))
        dd = jnp.sqrt((X - xl) ** 2 + (Y - yl) ** 2 + (Z - zl) ** 2 + EPS)
        d = jnp.minimum(d, jnp.where(valid, dd, jnp.float32(-1.0)))
        put = iota_m == i
        idxa = jnp.where(put, last, idxa)
        cxa = jnp.where(put, xl, cxa)
        cya = jnp.where(put, yl, cya)
        cza = jnp.where(put, zl, cza)
        mx = jnp.max(d)
        nxt = jnp.min(jnp.where(d == mx, iota_n, n_lanes)).astype(jnp.int32)
        return d, nxt, idxa, cxa, cya, cza

    d, _, idxa, cxa, cya, cza = jax.lax.fori_loop(
        0, m, body, (d0, jnp.int32(0), idx0, z0, z0, z0))
    return idxa, cxa, cya, cza, jnp.max(d)


def _fps_kernel(xs_ref, ys_ref, zs_ref,
                idx0_ref, c0x_ref, c0y_ref, c0z_ref,
                idx1_ref, c1x_ref, c1y_ref, c1z_ref,
                idx2_ref, c2x_ref, c2y_ref, c2z_ref,
                scal_ref):
    X0, Y0, Z0 = xs_ref[...], ys_ref[...], zs_ref[...]
    idx0, c0x, c0y, c0z, maxd0 = _fps_level(X0, Y0, Z0, 4096, 512, 4)
    idx0_ref[...] = idx0
    c0x_ref[...] = c0x
    c0y_ref[...] = c0y
    c0z_ref[...] = c0z

    idx1, c1x, c1y, c1z, maxd1 = _fps_level(c0x, c0y, c0z, 512, 64, 1)
    idx1_ref[...] = idx1
    c1x_ref[...] = c1x
    c1y_ref[...] = c1y
    c1z_ref[...] = c1z

    idx2, c2x, c2y, c2z, maxd2 = _fps_level(c1x, c1y, c1z, 64, 16, 1)
    idx2_ref[...] = idx2
    c2x_ref[...] = c2x
    c2y_ref[...] = c2y
    c2z_ref[...] = c2z

    iota = _flat_iota((1, 128))
    m8 = iota < 16
    gx = jnp.sum(jnp.where(m8, c2x, 0.0)) / 16.0
    gy = jnp.sum(jnp.where(m8, c2y, 0.0)) / 16.0
    gz = jnp.sum(jnp.where(m8, c2z, 0.0)) / 16.0
    dc = jnp.sqrt((c2x - gx) ** 2 + (c2y - gy) ** 2 + (c2z - gz) ** 2 + EPS)
    dc = jnp.where(m8, dc, jnp.float32(BIG))
    mn = jnp.min(dc)
    cidx = jnp.min(jnp.where(dc == mn, iota, 128)).astype(jnp.int32)
    csel = iota == cidx
    ccx = jnp.sum(jnp.where(csel, c2x, 0.0))
    ccy = jnp.sum(jnp.where(csel, c2y, 0.0))
    ccz = jnp.sum(jnp.where(csel, c2z, 0.0))

    lane = iota
    scal = jnp.zeros((1, 128), jnp.float32)
    for pos_i, val in enumerate([maxd0, maxd1, maxd2, ccx, ccy, ccz]):
        scal = jnp.where(lane == pos_i, val, scal)
    scal_ref[...] = scal


def _topk_kernel(px_ref, py_ref, pz_ref, cx_ref, cy_ref, cz_ref,
                 cidx_ref, maxd_ref, nbrT_ref, *, n):
    px, py, pz = px_ref[...], py_ref[...], pz_ref[...]
    cx, cy, cz = cx_ref[...], cy_ref[...], cz_ref[...]
    D = jnp.sqrt((px - cx) ** 2 + (py - cy) ** 2 + (pz - cz) ** 2 + EPS)
    radius = 2.0 * maxd_ref[0]
    row_iota = jax.lax.broadcasted_iota(jnp.int32, D.shape, 0)
    cidx = cidx_ref[...]
    for t in range(KG):
        vmin = jnp.min(D, axis=0, keepdims=True)
        sel = jnp.min(jnp.where(D == vmin, row_iota, n),
                      axis=0, keepdims=True)
        nbrT_ref[t:t + 1, :] = jnp.where(vmin <= radius, sel, cidx)
        D = jnp.where(row_iota == sel, jnp.float32(BIG), D)


def _sc_gather(table, idx):
    v_rows, d = table.shape
    b = idx.shape[0]
    info = plsc.get_sparse_core_info()
    nw = info.num_cores * info.num_subcores
    b_per_w = b // nw
    chunk = min(b_per_w, 128)
    nchunk = b_per_w // chunk
    idx3 = idx.reshape(nw, nchunk, chunk)
    mesh = plsc.VectorSubcoreMesh(core_axis_name="c", subcore_axis_name="s")

    @functools.partial(
        pl.kernel, mesh=mesh,
        out_type=jax.ShapeDtypeStruct((b, d), jnp.float32),
        scratch_types=[
            pltpu.VMEM((nchunk, chunk), jnp.int32),
            pltpu.VMEM((chunk, d), jnp.float32),
            pltpu.VMEM((chunk, d), jnp.float32),
            pltpu.SemaphoreType.DMA,
            pltpu.SemaphoreType.DMA,
        ],
    )
    def gk(table_hbm, idx_hbm, out_hbm, idx_v, buf0, buf1, sem0, sem1):
        wid = jax.lax.axis_index("s") * info.num_cores + jax.lax.axis_index("c")
        base = wid * b_per_w
        pltpu.sync_copy(idx_hbm.at[wid], idx_v)
        bufs = (buf0, buf1)
        sems = (sem0, sem1)
        handles = [None, None]
        for c in range(nchunk):
            handles[c % 2] = pltpu.async_copy(
                table_hbm.at[idx_v.at[c]], bufs[c % 2], sems[c % 2])
            if c > 0:
                handles[(c - 1) % 2].wait()
                pltpu.sync_copy(
                    bufs[(c - 1) % 2],
                    out_hbm.at[pl.ds(base + (c - 1) * chunk, chunk)])
        handles[(nchunk - 1) % 2].wait()
        pltpu.sync_copy(
            bufs[(nchunk - 1) % 2],
            out_hbm.at[pl.ds(base + (nchunk - 1) * chunk, chunk)])

    return gk(table, idx3)


def _sa_kernel(rows_ref, cpad_ref,
               w1_ref, b1_ref, w2_ref, b2_ref, w3_ref, b3_ref,
               out_ref, *, rows):
    feat = rows_ref[...] - cpad_ref[...]
    h = jnp.tanh(_mm(feat, w1_ref[...]) + b1_ref[...])
    h = jnp.tanh(_mm(h, w2_ref[...]) + b2_ref[...])
    h = jnp.tanh(_mm(h, w3_ref[...]) + b3_ref[...])
    m = rows // KG
    acc = h[0:m, :]
    for t in range(1, KG):
        acc = jnp.maximum(acc, h[t * m:(t + 1) * m, :])
    out_ref[...] = acc


def _global_kernel(data_ref, crow_ref, g1_ref, bg1_ref, g2_ref, bg2_ref,
                   g3_ref, bg3_ref, pospad_ref, d1a_ref, d1b_ref, bd1_ref,
                   d2_ref, bd2_ref, out_ref):
    f = data_ref[...] - crow_ref[...]
    h = jnp.tanh(_mm(f, g1_ref[...]) + bg1_ref[...])
    h = jnp.tanh(_mm(h, g2_ref[...]) + bg2_ref[...])
    h = jnp.tanh(_mm(h, g3_ref[...]) + bg3_ref[...])
    pooled = jnp.max(h, axis=0, keepdims=True)
    h2 = jnp.tanh(_mm(pospad_ref[...], d1a_ref[...])
                  + _mm(pooled, d1b_ref[...]) + bd1_ref[...])
    out_ref[...] = jnp.tanh(_mm(h2, d2_ref[...]) + bd2_ref[...])


def _interp_kernel(ux_ref, uy_ref, uz_ref, kx_ref, ky_ref, kz_ref,
                   cur_ref, pospad_ref, w1a_ref, w1b_ref, b1_ref,
                   w2_ref, b2_ref, out_ref, *, k_valid):
    ux, uy, uz = ux_ref[...], uy_ref[...], uz_ref[...]
    kx, ky, kz = kx_ref[...], ky_ref[...], kz_ref[...]
    D = jnp.sqrt((ux - kx) ** 2 + (uy - ky) ** 2 + (uz - kz) ** 2 + EPS)
    kpad = D.shape[1]
    col_iota = jax.lax.broadcasted_iota(jnp.int32, D.shape, 1)
    D = jnp.where(col_iota < k_valid, D, jnp.float32(BIG))
    vals, sels = [], []
    for _ in range(3):
        vmin = jnp.min(D, axis=1, keepdims=True)
        sel = jnp.min(jnp.where(D == vmin, col_iota, kpad),
                      axis=1, keepdims=True)
        vals.append(vmin)
        sels.append(sel)
        D = jnp.where(col_iota == sel, jnp.float32(BIG), D)
    ws = [1.0 / jnp.maximum(v, 1e-10) for v in vals]
    wsum = ws[0] + ws[1] + ws[2]
    Wmat = jnp.zeros_like(D)
    for t in range(3):
        Wmat = Wmat + jnp.where(col_iota == sels[t], ws[t] / wsum, 0.0)
    interp = _mm(Wmat, cur_ref[...])
    h = jnp.tanh(_mm(pospad_ref[...], w1a_ref[...])
                 + _mm(interp, w1b_ref[...]) + b1_ref[...])
    out_ref[...] = jnp.tanh(_mm(h, w2_ref[...]) + b2_ref[...])


def _pad_cols(a, width):
    return jnp.concatenate(
        [a, jnp.zeros((a.shape[0], width - a.shape[1]), a.dtype)], axis=1)


def _pad_rows(a, height):
    return jnp.concatenate(
        [a, jnp.zeros((height - a.shape[0], a.shape[1]), a.dtype)], axis=0)


def _f32(shape):
    return jax.ShapeDtypeStruct(shape, jnp.float32)


def _i32(shape):
    return jax.ShapeDtypeStruct(shape, jnp.int32)


def _run_topk(pcols, crow, cidxrow, maxd, n, mpad):
    fn = functools.partial(_topk_kernel, n=n)
    return pl.pallas_call(
        fn,
        in_specs=[pl.BlockSpec(memory_space=pltpu.VMEM)] * 6
        + [pl.BlockSpec(memory_space=pltpu.VMEM),
           pl.BlockSpec(memory_space=pltpu.SMEM)],
        out_shape=_i32((KG, mpad)),
    )(pcols[0], pcols[1], pcols[2], crow[0], crow[1], crow[2],
      cidxrow, maxd)


def _run_sa(nbr_flat, data, cpad, layers, rows, cout):
    w1, b1 = layers[0]
    w2, b2 = layers[1]
    w3, b3 = layers[2]
    cpadw = data.shape[1]
    w1p = _pad_rows(w1, cpadw)
    grp = _sc_gather(data, nbr_flat)
    fn = functools.partial(_sa_kernel, rows=rows)
    return pl.pallas_call(
        fn,
        out_shape=_f32((rows // KG, cout)),
    )(grp, cpad,
      w1p, b1.reshape(1, -1), w2, b2.reshape(1, -1), w3, b3.reshape(1, -1))


def _run_interp(upcols, krow, cur_pad, pospad, layers, k_valid, uk, cout):
    (w1, b1), (w2, b2) = layers
    w1a = _pad_rows(w1[:3], 128)
    w1b = w1[3:]
    fn = functools.partial(_interp_kernel, k_valid=k_valid)
    return pl.pallas_call(
        fn,
        out_shape=_f32((uk, cout)),
    )(upcols[0], upcols[1], upcols[2], krow[0], krow[1], krow[2],
      cur_pad, pospad, w1a, w1b, b1.reshape(1, -1), w2, b2.reshape(1, -1))


def kernel(inputs, params):
    pos = inputs[:, :3]
    xs = inputs[:, 0].reshape(32, 128)
    ys = inputs[:, 1].reshape(32, 128)
    zs = inputs[:, 2].reshape(32, 128)

    a_out = pl.pallas_call(
        _fps_kernel,
        out_shape=[_i32((4, 128)), _f32((4, 128)), _f32((4, 128)),
                   _f32((4, 128)),
                   _i32((1, 128)), _f32((1, 128)), _f32((1, 128)),
                   _f32((1, 128)),
                   _i32((1, 128)), _f32((1, 128)), _f32((1, 128)),
                   _f32((1, 128)),
                   _f32((1, 128))],
    )(xs, ys, zs)
    (idx0, c0x, c0y, c0z, idx1, c1x, c1y, c1z,
     idx2, c2x, c2y, c2z, scal) = a_out

    conv = params["conv"]
    conv_re = params["conv_re"]

    pcols0 = [pos[:, 0:1], pos[:, 1:2], pos[:, 2:3]]
    crow0 = [c0x.reshape(1, 512), c0y.reshape(1, 512), c0z.reshape(1, 512)]
    nbrT0 = _run_topk(pcols0, crow0, idx0.reshape(1, 512),
                      scal[0, 0:1], 4096, 512)
    cps0 = jnp.stack([c0x.reshape(512), c0y.reshape(512),
                      c0z.reshape(512)], axis=1)
    cpad0 = jnp.tile(_pad_cols(cps0, 128), (KG, 1))
    pooled0 = _run_sa(nbrT0.reshape(-1), _pad_cols(inputs, 128), cpad0,
                      conv[0], KG * 512, 128)

    data1 = _pad_cols(jnp.concatenate([cps0, pooled0], axis=1), 256)
    pcols1 = [cps0[:, 0:1], cps0[:, 1:2], cps0[:, 2:3]]
    crow1 = [c1x, c1y, c1z]
    nbrT1 = _run_topk(pcols1, crow1, idx1, scal[0, 1:2], 512, 128)
    cps1 = jnp.stack([c1x[0, :64], c1y[0, :64], c1z[0, :64]], axis=1)
    cpad1 = jnp.tile(_pad_cols(cps1, 256), (KG, 1))
    pooled1 = _run_sa(nbrT1[:, :64].reshape(-1), data1, cpad1,
                      conv[1], KG * 64, 256)

    data2 = _pad_cols(jnp.concatenate([cps1, pooled1], axis=1), 384)
    pcols2 = [cps1[:, 0:1], cps1[:, 1:2], cps1[:, 2:3]]
    crow2 = [c2x, c2y, c2z]
    nbrT2 = _run_topk(pcols2, crow2, idx2, scal[0, 2:3], 64, 128)
    cps2 = jnp.stack([c2x[0, :16], c2y[0, :16], c2z[0, :16]], axis=1)
    cpad2 = jnp.tile(_pad_cols(cps2, 384), (KG, 1))
    pooled2 = _run_sa(nbrT2[:, :16].reshape(-1), data2, cpad2,
                      conv[2], KG * 16, 512)

    data3 = _pad_cols(jnp.concatenate([cps2, pooled2], axis=1), 640)
    crow = _pad_cols(scal[0:1, 3:6], 640)
    (g1, bg1), (g2, bg2), (g3, bg3) = conv[3]
    (rd1, rb1), (rd2, rb2) = conv_re[0]
    cur3 = pl.pallas_call(
        _global_kernel,
        out_shape=_f32((16, 512)),
    )(data3, crow, _pad_rows(g1, 640), bg1.reshape(1, -1),
      g2, bg2.reshape(1, -1), g3, bg3.reshape(1, -1),
      _pad_cols(cps2, 128), _pad_rows(rd1[:3], 128), rd1[3:],
      rb1.reshape(1, -1), rd2, rb2.reshape(1, -1))

    cur2 = _run_interp(pcols2, crow2, _pad_rows(cur3, 128),
                       _pad_cols(cps1, 128), conv_re[1], 16, 64, 256)

    cur1 = _run_interp(pcols1, crow1, _pad_rows(cur2, 128),
                       _pad_cols(cps0, 128), conv_re[2], 64, 512, 128)

    out = _run_interp(pcols0, crow0, cur1,
                      _pad_cols(pos, 128), conv_re[3], 512, 4096, 128)
    return out


# speedup vs baseline: 6.9987x; 1.2114x over previous
import functools

import jax
import jax.numpy as jnp
from jax.experimental import pallas as pl
from jax.experimental.pallas import tpu as pltpu
from jax.experimental.pallas import tpu_sc as plsc

EPS = 1e-12
KG = 32
BIG = 1e30


def _mm(a, b):
    return jax.lax.dot_general(
        a, b, (((1,), (0,)), ((), ())),
        precision=jax.lax.Precision.HIGHEST,
        preferred_element_type=jnp.float32)


def _flat_iota(shape):
    r = jax.lax.broadcasted_iota(jnp.int32, shape, 0)
    c = jax.lax.broadcasted_iota(jnp.int32, shape, 1)
    return r * shape[1] + c


def _fps_kernel(xs_ref, ys_ref, zs_ref, xss_ref, yss_ref, zss_ref,
                idx0_ref, c0x_ref, c0y_ref, c0z_ref,
                idx1_ref, c1x_ref, c1y_ref, c1z_ref,
                idx2_ref, c2x_ref, c2y_ref, c2z_ref,
                scal_ref, sel_ref):
    X0, Y0, Z0 = xs_ref[...], ys_ref[...], zs_ref[...]
    iota_n = _flat_iota((32, 128))
    iota_m = _flat_iota((4, 128))
    z4 = jnp.zeros((4, 128), jnp.float32)

    def body(i, carry):
        d, last, idxa, cxa, cya, cza = carry
        sel_ref[i] = last
        xl = xss_ref[last]
        yl = yss_ref[last]
        zl = zss_ref[last]
        dd = jnp.sqrt((X0 - xl) ** 2 + (Y0 - yl) ** 2 + (Z0 - zl) ** 2 + EPS)
        d = jnp.minimum(d, dd)
        put = iota_m == i
        idxa = jnp.where(put, last, idxa)
        cxa = jnp.where(put, xl, cxa)
        cya = jnp.where(put, yl, cya)
        cza = jnp.where(put, zl, cza)
        mx = jnp.max(d)
        nxt = jnp.min(jnp.where(d == mx, iota_n, 4096)).astype(jnp.int32)
        return d, nxt, idxa, cxa, cya, cza

    d0f = jnp.full((32, 128), 1e10, jnp.float32)
    d, _, idx0, c0x, c0y, c0z = jax.lax.fori_loop(
        0, 512, body, (d0f, jnp.int32(0), jnp.zeros((4, 128), jnp.int32),
                       z4, z4, z4))
    maxd0 = jnp.max(d)
    idx0_ref[...] = idx0
    c0x_ref[...] = c0x
    c0y_ref[...] = c0y
    c0z_ref[...] = c0z

    lane = _flat_iota((1, 128))

    def body1(i, d1):
        si = sel_ref[i]
        xl = xss_ref[si]
        yl = yss_ref[si]
        zl = zss_ref[si]
        dd = jnp.sqrt((c0x - xl) ** 2 + (c0y - yl) ** 2 + (c0z - zl) ** 2
                      + EPS)
        return jnp.minimum(d1, dd)

    d1 = jax.lax.fori_loop(0, 64, body1, jnp.full((4, 128), 1e10, jnp.float32))
    maxd1 = jnp.max(d1)
    idx1_ref[...] = lane
    m64 = lane < 64
    c1x = jnp.where(m64, c0x[0:1, :], 0.0)
    c1y = jnp.where(m64, c0y[0:1, :], 0.0)
    c1z = jnp.where(m64, c0z[0:1, :], 0.0)
    c1x_ref[...] = c1x
    c1y_ref[...] = c1y
    c1z_ref[...] = c1z

    def body2(i, d2):
        si = sel_ref[i]
        xl = xss_ref[si]
        yl = yss_ref[si]
        zl = zss_ref[si]
        dd = jnp.sqrt((c1x - xl) ** 2 + (c1y - yl) ** 2 + (c1z - zl) ** 2
                      + EPS)
        return jnp.minimum(d2, dd)

    d2 = jax.lax.fori_loop(0, 16, body2, jnp.full((1, 128), 1e10, jnp.float32))
    maxd2 = jnp.max(jnp.where(m64, d2, jnp.float32(-1.0)))
    idx2_ref[...] = lane
    m16 = lane < 16
    c2x = jnp.where(m16, c1x, 0.0)
    c2y = jnp.where(m16, c1y, 0.0)
    c2z = jnp.where(m16, c1z, 0.0)
    c2x_ref[...] = c2x
    c2y_ref[...] = c2y
    c2z_ref[...] = c2z

    iota = _flat_iota((1, 128))
    m8 = iota < 16
    gx = jnp.sum(jnp.where(m8, c2x, 0.0)) / 16.0
    gy = jnp.sum(jnp.where(m8, c2y, 0.0)) / 16.0
    gz = jnp.sum(jnp.where(m8, c2z, 0.0)) / 16.0
    dc = jnp.sqrt((c2x - gx) ** 2 + (c2y - gy) ** 2 + (c2z - gz) ** 2 + EPS)
    dc = jnp.where(m8, dc, jnp.float32(BIG))
    mn = jnp.min(dc)
    cidx = jnp.min(jnp.where(dc == mn, iota, 128)).astype(jnp.int32)
    csel = iota == cidx
    ccx = jnp.sum(jnp.where(csel, c2x, 0.0))
    ccy = jnp.sum(jnp.where(csel, c2y, 0.0))
    ccz = jnp.sum(jnp.where(csel, c2z, 0.0))

    lane = iota
    scal = jnp.zeros((1, 128), jnp.float32)
    for pos_i, val in enumerate([maxd0, maxd1, maxd2, ccx, ccy, ccz]):
        scal = jnp.where(lane == pos_i, val, scal)
    scal_ref[...] = scal


def _topk_kernel(px_ref, py_ref, pz_ref, cx_ref, cy_ref, cz_ref,
                 cidx_ref, maxd_ref, nbrT_ref, *, n):
    px, py, pz = px_ref[...], py_ref[...], pz_ref[...]
    cx, cy, cz = cx_ref[...], cy_ref[...], cz_ref[...]
    D = jnp.sqrt((px - cx) ** 2 + (py - cy) ** 2 + (pz - cz) ** 2 + EPS)
    radius = 2.0 * maxd_ref[0]
    row_iota = jax.lax.broadcasted_iota(jnp.int32, D.shape, 0)
    cidx = cidx_ref[...]
    for t in range(KG):
        vmin = jnp.min(D, axis=0, keepdims=True)
        sel = jnp.min(jnp.where(D == vmin, row_iota, n),
                      axis=0, keepdims=True)
        nbrT_ref[t:t + 1, :] = jnp.where(vmin <= radius, sel, cidx)
        D = jnp.where(row_iota == sel, jnp.float32(BIG), D)


def _sc_gather(table, idx):
    v_rows, d = table.shape
    b = idx.shape[0]
    info = plsc.get_sparse_core_info()
    nw = info.num_cores * info.num_subcores
    b_per_w = b // nw
    chunk = min(b_per_w, 128)
    nchunk = b_per_w // chunk
    idx3 = idx.reshape(nw, nchunk, chunk)
    mesh = plsc.VectorSubcoreMesh(core_axis_name="c", subcore_axis_name="s")

    @functools.partial(
        pl.kernel, mesh=mesh,
        out_type=jax.ShapeDtypeStruct((b, d), jnp.float32),
        scratch_types=[
            pltpu.VMEM((nchunk, chunk), jnp.int32),
            pltpu.VMEM((chunk, d), jnp.float32),
            pltpu.VMEM((chunk, d), jnp.float32),
            pltpu.SemaphoreType.DMA,
            pltpu.SemaphoreType.DMA,
        ],
    )
    def gk(table_hbm, idx_hbm, out_hbm, idx_v, buf0, buf1, sem0, sem1):
        wid = jax.lax.axis_index("s") * info.num_cores + jax.lax.axis_index("c")
        base = wid * b_per_w
        pltpu.sync_copy(idx_hbm.at[wid], idx_v)
        bufs = (buf0, buf1)
        sems = (sem0, sem1)
        handles = [None, None]
        for c in range(nchunk):
            handles[c % 2] = pltpu.async_copy(
                table_hbm.at[idx_v.at[c]], bufs[c % 2], sems[c % 2])
            if c > 0:
                handles[(c - 1) % 2].wait()
                pltpu.sync_copy(
                    bufs[(c - 1) % 2],
                    out_hbm.at[pl.ds(base + (c - 1) * chunk, chunk)])
        handles[(nchunk - 1) % 2].wait()
        pltpu.sync_copy(
            bufs[(nchunk - 1) % 2],
            out_hbm.at[pl.ds(base + (nchunk - 1) * chunk, chunk)])

    return gk(table, idx3)


def _sa_kernel(rows_ref, cpad_ref,
               w1_ref, b1_ref, w2_ref, b2_ref, w3_ref, b3_ref,
               out_ref, *, rows):
    feat = rows_ref[...] - cpad_ref[...]
    h = jnp.tanh(_mm(feat, w1_ref[...]) + b1_ref[...])
    h = jnp.tanh(_mm(h, w2_ref[...]) + b2_ref[...])
    h = jnp.tanh(_mm(h, w3_ref[...]) + b3_ref[...])
    m = rows // KG
    acc = h[0:m, :]
    for t in range(1, KG):
        acc = jnp.maximum(acc, h[t * m:(t + 1) * m, :])
    out_ref[...] = acc


def _global_kernel(data_ref, crow_ref, g1_ref, bg1_ref, g2_ref, bg2_ref,
                   g3_ref, bg3_ref, pospad_ref, d1a_ref, d1b_ref, bd1_ref,
                   d2_ref, bd2_ref, out_ref):
    f = data_ref[...] - crow_ref[...]
    h = jnp.tanh(_mm(f, g1_ref[...]) + bg1_ref[...])
    h = jnp.tanh(_mm(h, g2_ref[...]) + bg2_ref[...])
    h = jnp.tanh(_mm(h, g3_ref[...]) + bg3_ref[...])
    pooled = jnp.max(h, axis=0, keepdims=True)
    h2 = jnp.tanh(_mm(pospad_ref[...], d1a_ref[...])
                  + _mm(pooled, d1b_ref[...]) + bd1_ref[...])
    out_ref[...] = jnp.tanh(_mm(h2, d2_ref[...]) + bd2_ref[...])


def _interp_kernel(ux_ref, uy_ref, uz_ref, kx_ref, ky_ref, kz_ref,
                   cur_ref, pospad_ref, w1a_ref, w1b_ref, b1_ref,
                   w2_ref, b2_ref, out_ref, *, k_valid):
    ux, uy, uz = ux_ref[...], uy_ref[...], uz_ref[...]
    kx, ky, kz = kx_ref[...], ky_ref[...], kz_ref[...]
    D = jnp.sqrt((ux - kx) ** 2 + (uy - ky) ** 2 + (uz - kz) ** 2 + EPS)
    kpad = D.shape[1]
    col_iota = jax.lax.broadcasted_iota(jnp.int32, D.shape, 1)
    D = jnp.where(col_iota < k_valid, D, jnp.float32(BIG))
    vals, sels = [], []
    for _ in range(3):
        vmin = jnp.min(D, axis=1, keepdims=True)
        sel = jnp.min(jnp.where(D == vmin, col_iota, kpad),
                      axis=1, keepdims=True)
        vals.append(vmin)
        sels.append(sel)
        D = jnp.where(col_iota == sel, jnp.float32(BIG), D)
    ws = [1.0 / jnp.maximum(v, 1e-10) for v in vals]
    wsum = ws[0] + ws[1] + ws[2]
    Wmat = jnp.zeros_like(D)
    for t in range(3):
        Wmat = Wmat + jnp.where(col_iota == sels[t], ws[t] / wsum, 0.0)
    interp = _mm(Wmat, cur_ref[...])
    h = jnp.tanh(_mm(pospad_ref[...], w1a_ref[...])
                 + _mm(interp, w1b_ref[...]) + b1_ref[...])
    out_ref[...] = jnp.tanh(_mm(h, w2_ref[...]) + b2_ref[...])


def _pad_cols(a, width):
    return jnp.concatenate(
        [a, jnp.zeros((a.shape[0], width - a.shape[1]), a.dtype)], axis=1)


def _pad_rows(a, height):
    return jnp.concatenate(
        [a, jnp.zeros((height - a.shape[0], a.shape[1]), a.dtype)], axis=0)


def _f32(shape):
    return jax.ShapeDtypeStruct(shape, jnp.float32)


def _i32(shape):
    return jax.ShapeDtypeStruct(shape, jnp.int32)


def _run_topk(pcols, crow, cidxrow, maxd, n, mpad):
    fn = functools.partial(_topk_kernel, n=n)
    return pl.pallas_call(
        fn,
        in_specs=[pl.BlockSpec(memory_space=pltpu.VMEM)] * 6
        + [pl.BlockSpec(memory_space=pltpu.VMEM),
           pl.BlockSpec(memory_space=pltpu.SMEM)],
        out_shape=_i32((KG, mpad)),
    )(pcols[0], pcols[1], pcols[2], crow[0], crow[1], crow[2],
      cidxrow, maxd)


def _run_sa(nbr_flat, data, cpad, layers, rows, cout):
    w1, b1 = layers[0]
    w2, b2 = layers[1]
    w3, b3 = layers[2]
    cpadw = data.shape[1]
    w1p = _pad_rows(w1, cpadw)
    grp = _sc_gather(data, nbr_flat)
    fn = functools.partial(_sa_kernel, rows=rows)
    return pl.pallas_call(
        fn,
        out_shape=_f32((rows // KG, cout)),
    )(grp, cpad,
      w1p, b1.reshape(1, -1), w2, b2.reshape(1, -1), w3, b3.reshape(1, -1))


def _run_interp(upcols, krow, cur_pad, pospad, layers, k_valid, uk, cout):
    (w1, b1), (w2, b2) = layers
    w1a = _pad_rows(w1[:3], 128)
    w1b = w1[3:]
    fn = functools.partial(_interp_kernel, k_valid=k_valid)
    return pl.pallas_call(
        fn,
        out_shape=_f32((uk, cout)),
    )(upcols[0], upcols[1], upcols[2], krow[0], krow[1], krow[2],
      cur_pad, pospad, w1a, w1b, b1.reshape(1, -1), w2, b2.reshape(1, -1))


def kernel(inputs, params):
    pos = inputs[:, :3]
    xs = inputs[:, 0].reshape(32, 128)
    ys = inputs[:, 1].reshape(32, 128)
    zs = inputs[:, 2].reshape(32, 128)

    a_out = pl.pallas_call(
        _fps_kernel,
        in_specs=[pl.BlockSpec(memory_space=pltpu.VMEM)] * 3
        + [pl.BlockSpec(memory_space=pltpu.SMEM)] * 3,
        out_shape=[_i32((4, 128)), _f32((4, 128)), _f32((4, 128)),
                   _f32((4, 128)),
                   _i32((1, 128)), _f32((1, 128)), _f32((1, 128)),
                   _f32((1, 128)),
                   _i32((1, 128)), _f32((1, 128)), _f32((1, 128)),
                   _f32((1, 128)),
                   _f32((1, 128))],
        scratch_shapes=[pltpu.SMEM((512,), jnp.int32)],
    )(xs, ys, zs, inputs[:, 0], inputs[:, 1], inputs[:, 2])
    (idx0, c0x, c0y, c0z, idx1, c1x, c1y, c1z,
     idx2, c2x, c2y, c2z, scal) = a_out

    conv = params["conv"]
    conv_re = params["conv_re"]

    pcols0 = [pos[:, 0:1], pos[:, 1:2], pos[:, 2:3]]
    crow0 = [c0x.reshape(1, 512), c0y.reshape(1, 512), c0z.reshape(1, 512)]
    nbrT0 = _run_topk(pcols0, crow0, idx0.reshape(1, 512),
                      scal[0, 0:1], 4096, 512)
    cps0 = jnp.stack([c0x.reshape(512), c0y.reshape(512),
                      c0z.reshape(512)], axis=1)
    cpad0 = jnp.tile(_pad_cols(cps0, 128), (KG, 1))
    pooled0 = _run_sa(nbrT0.reshape(-1), _pad_cols(inputs, 128), cpad0,
                      conv[0], KG * 512, 128)

    data1 = _pad_cols(jnp.concatenate([cps0, pooled0], axis=1), 256)
    pcols1 = [cps0[:, 0:1], cps0[:, 1:2], cps0[:, 2:3]]
    crow1 = [c1x, c1y, c1z]
    nbrT1 = _run_topk(pcols1, crow1, idx1, scal[0, 1:2], 512, 128)
    cps1 = jnp.stack([c1x[0, :64], c1y[0, :64], c1z[0, :64]], axis=1)
    cpad1 = jnp.tile(_pad_cols(cps1, 256), (KG, 1))
    pooled1 = _run_sa(nbrT1[:, :64].reshape(-1), data1, cpad1,
                      conv[1], KG * 64, 256)

    data2 = _pad_cols(jnp.concatenate([cps1, pooled1], axis=1), 384)
    pcols2 = [cps1[:, 0:1], cps1[:, 1:2], cps1[:, 2:3]]
    crow2 = [c2x, c2y, c2z]
    nbrT2 = _run_topk(pcols2, crow2, idx2, scal[0, 2:3], 64, 128)
    cps2 = jnp.stack([c2x[0, :16], c2y[0, :16], c2z[0, :16]], axis=1)
    cpad2 = jnp.tile(_pad_cols(cps2, 384), (KG, 1))
    pooled2 = _run_sa(nbrT2[:, :16].reshape(-1), data2, cpad2,
                      conv[2], KG * 16, 512)

    data3 = _pad_cols(jnp.concatenate([cps2, pooled2], axis=1), 640)
    crow = _pad_cols(scal[0:1, 3:6], 640)
    (g1, bg1), (g2, bg2), (g3, bg3) = conv[3]
    (rd1, rb1), (rd2, rb2) = conv_re[0]
    cur3 = pl.pallas_call(
        _global_kernel,
        out_shape=_f32((16, 512)),
    )(data3, crow, _pad_rows(g1, 640), bg1.reshape(1, -1),
      g2, bg2.reshape(1, -1), g3, bg3.reshape(1, -1),
      _pad_cols(cps2, 128), _pad_rows(rd1[:3], 128), rd1[3:],
      rb1.reshape(1, -1), rd2, rb2.reshape(1, -1))

    cur2 = _run_interp(pcols2, crow2, _pad_rows(cur3, 128),
                       _pad_cols(cps1, 128), conv_re[1], 16, 64, 256)

    cur1 = _run_interp(pcols1, crow1, _pad_rows(cur2, 128),
                       _pad_cols(cps0, 128), conv_re[2], 64, 512, 128)

    out = _run_interp(pcols0, crow0, cur1,
                      _pad_cols(pos, 128), conv_re[3], 512, 4096, 128)
    return out


# speedup vs baseline: 8.1496x; 1.1644x over previous
import functools

import jax
import jax.numpy as jnp
from jax.experimental import pallas as pl
from jax.experimental.pallas import tpu as pltpu
from jax.experimental.pallas import tpu_sc as plsc

EPS = 1e-12
KG = 32
BIG = 1e30


def _mm(a, b):
    return jax.lax.dot_general(
        a, b, (((1,), (0,)), ((), ())),
        precision=jax.lax.Precision.DEFAULT,
        preferred_element_type=jnp.float32)


def _flat_iota(shape):
    r = jax.lax.broadcasted_iota(jnp.int32, shape, 0)
    c = jax.lax.broadcasted_iota(jnp.int32, shape, 1)
    return r * shape[1] + c


def _fps_kernel(xs_ref, ys_ref, zs_ref, xss_ref, yss_ref, zss_ref,
                idx0_ref, c0x_ref, c0y_ref, c0z_ref,
                idx1_ref, c1x_ref, c1y_ref, c1z_ref,
                idx2_ref, c2x_ref, c2y_ref, c2z_ref,
                scal_ref, sel_ref):
    X0, Y0, Z0 = xs_ref[...], ys_ref[...], zs_ref[...]
    iota_n = _flat_iota((32, 128))
    iota_m = _flat_iota((4, 128))
    z4 = jnp.zeros((4, 128), jnp.float32)

    def body(i, carry):
        d, last, idxa, cxa, cya, cza = carry
        sel_ref[i] = last
        xl = xss_ref[last]
        yl = yss_ref[last]
        zl = zss_ref[last]
        dd = jnp.sqrt((X0 - xl) ** 2 + (Y0 - yl) ** 2 + (Z0 - zl) ** 2 + EPS)
        d = jnp.minimum(d, dd)
        put = iota_m == i
        idxa = jnp.where(put, last, idxa)
        cxa = jnp.where(put, xl, cxa)
        cya = jnp.where(put, yl, cya)
        cza = jnp.where(put, zl, cza)
        mx = jnp.max(d)
        nxt = jnp.min(jnp.where(d == mx, iota_n, 4096)).astype(jnp.int32)
        return d, nxt, idxa, cxa, cya, cza

    d0f = jnp.full((32, 128), 1e10, jnp.float32)
    d, _, idx0, c0x, c0y, c0z = jax.lax.fori_loop(
        0, 512, body, (d0f, jnp.int32(0), jnp.zeros((4, 128), jnp.int32),
                       z4, z4, z4))
    maxd0 = jnp.max(d)
    idx0_ref[...] = idx0
    c0x_ref[...] = c0x
    c0y_ref[...] = c0y
    c0z_ref[...] = c0z

    lane = _flat_iota((1, 128))

    def body1(i, d1):
        si = sel_ref[i]
        xl = xss_ref[si]
        yl = yss_ref[si]
        zl = zss_ref[si]
        dd = jnp.sqrt((c0x - xl) ** 2 + (c0y - yl) ** 2 + (c0z - zl) ** 2
                      + EPS)
        return jnp.minimum(d1, dd)

    d1 = jax.lax.fori_loop(0, 64, body1, jnp.full((4, 128), 1e10, jnp.float32))
    maxd1 = jnp.max(d1)
    idx1_ref[...] = lane
    m64 = lane < 64
    c1x = jnp.where(m64, c0x[0:1, :], 0.0)
    c1y = jnp.where(m64, c0y[0:1, :], 0.0)
    c1z = jnp.where(m64, c0z[0:1, :], 0.0)
    c1x_ref[...] = c1x
    c1y_ref[...] = c1y
    c1z_ref[...] = c1z

    def body2(i, d2):
        si = sel_ref[i]
        xl = xss_ref[si]
        yl = yss_ref[si]
        zl = zss_ref[si]
        dd = jnp.sqrt((c1x - xl) ** 2 + (c1y - yl) ** 2 + (c1z - zl) ** 2
                      + EPS)
        return jnp.minimum(d2, dd)

    d2 = jax.lax.fori_loop(0, 16, body2, jnp.full((1, 128), 1e10, jnp.float32))
    maxd2 = jnp.max(jnp.where(m64, d2, jnp.float32(-1.0)))
    idx2_ref[...] = lane
    m16 = lane < 16
    c2x = jnp.where(m16, c1x, 0.0)
    c2y = jnp.where(m16, c1y, 0.0)
    c2z = jnp.where(m16, c1z, 0.0)
    c2x_ref[...] = c2x
    c2y_ref[...] = c2y
    c2z_ref[...] = c2z

    iota = _flat_iota((1, 128))
    m8 = iota < 16
    gx = jnp.sum(jnp.where(m8, c2x, 0.0)) / 16.0
    gy = jnp.sum(jnp.where(m8, c2y, 0.0)) / 16.0
    gz = jnp.sum(jnp.where(m8, c2z, 0.0)) / 16.0
    dc = jnp.sqrt((c2x - gx) ** 2 + (c2y - gy) ** 2 + (c2z - gz) ** 2 + EPS)
    dc = jnp.where(m8, dc, jnp.float32(BIG))
    mn = jnp.min(dc)
    cidx = jnp.min(jnp.where(dc == mn, iota, 128)).astype(jnp.int32)
    csel = iota == cidx
    ccx = jnp.sum(jnp.where(csel, c2x, 0.0))
    ccy = jnp.sum(jnp.where(csel, c2y, 0.0))
    ccz = jnp.sum(jnp.where(csel, c2z, 0.0))

    lane = iota
    scal = jnp.zeros((1, 128), jnp.float32)
    for pos_i, val in enumerate([maxd0, maxd1, maxd2, ccx, ccy, ccz]):
        scal = jnp.where(lane == pos_i, val, scal)
    scal_ref[...] = scal


def _topk_body(px, py, pz, cx, cy, cz, cidx, radius, nbrT_ref, n):
    D = jnp.sqrt((px - cx) ** 2 + (py - cy) ** 2 + (pz - cz) ** 2 + EPS)
    row_iota = jax.lax.broadcasted_iota(jnp.int32, D.shape, 0)
    for t in range(KG):
        vmin = jnp.min(D, axis=0, keepdims=True)
        sel = jnp.min(jnp.where(D == vmin, row_iota, n),
                      axis=0, keepdims=True)
        nbrT_ref[t:t + 1, :] = jnp.where(vmin <= radius, sel, cidx)
        D = jnp.where(row_iota == sel, jnp.float32(BIG), D)


def _topk3_kernel(p0x_ref, p0y_ref, p0z_ref, c0x_ref, c0y_ref, c0z_ref,
                  i0_ref,
                  p1x_ref, p1y_ref, p1z_ref, c1x_ref, c1y_ref, c1z_ref,
                  i1_ref,
                  p2x_ref, p2y_ref, p2z_ref, c2x_ref, c2y_ref, c2z_ref,
                  i2_ref, maxds_ref,
                  nbrT0_ref, nbrT1_ref, nbrT2_ref):
    _topk_body(p0x_ref[...], p0y_ref[...], p0z_ref[...],
               c0x_ref[...], c0y_ref[...], c0z_ref[...],
               i0_ref[...], 2.0 * maxds_ref[0], nbrT0_ref, 4096)
    _topk_body(p1x_ref[...], p1y_ref[...], p1z_ref[...],
               c1x_ref[...], c1y_ref[...], c1z_ref[...],
               i1_ref[...], 2.0 * maxds_ref[1], nbrT1_ref, 512)
    _topk_body(p2x_ref[...], p2y_ref[...], p2z_ref[...],
               c2x_ref[...], c2y_ref[...], c2z_ref[...],
               i2_ref[...], 2.0 * maxds_ref[2], nbrT2_ref, 64)


def _sc_gather(table, idx):
    v_rows, d = table.shape
    b = idx.shape[0]
    info = plsc.get_sparse_core_info()
    nw = info.num_cores * info.num_subcores
    b_per_w = b // nw
    chunk = min(b_per_w, 128)
    nchunk = b_per_w // chunk
    idx3 = idx.reshape(nw, nchunk, chunk)
    mesh = plsc.VectorSubcoreMesh(core_axis_name="c", subcore_axis_name="s")

    @functools.partial(
        pl.kernel, mesh=mesh,
        out_type=jax.ShapeDtypeStruct((b, d), jnp.float32),
        scratch_types=[
            pltpu.VMEM((nchunk, chunk), jnp.int32),
            pltpu.VMEM((chunk, d), jnp.float32),
            pltpu.VMEM((chunk, d), jnp.float32),
            pltpu.SemaphoreType.DMA,
            pltpu.SemaphoreType.DMA,
        ],
    )
    def gk(table_hbm, idx_hbm, out_hbm, idx_v, buf0, buf1, sem0, sem1):
        wid = jax.lax.axis_index("s") * info.num_cores + jax.lax.axis_index("c")
        base = wid * b_per_w
        pltpu.sync_copy(idx_hbm.at[wid], idx_v)
        bufs = (buf0, buf1)
        sems = (sem0, sem1)
        handles = [None, None]
        for c in range(nchunk):
            handles[c % 2] = pltpu.async_copy(
                table_hbm.at[idx_v.at[c]], bufs[c % 2], sems[c % 2])
            if c > 0:
                handles[(c - 1) % 2].wait()
                pltpu.sync_copy(
                    bufs[(c - 1) % 2],
                    out_hbm.at[pl.ds(base + (c - 1) * chunk, chunk)])
        handles[(nchunk - 1) % 2].wait()
        pltpu.sync_copy(
            bufs[(nchunk - 1) % 2],
            out_hbm.at[pl.ds(base + (nchunk - 1) * chunk, chunk)])

    return gk(table, idx3)


def _sa_kernel(rows_ref, cpad_ref,
               w1_ref, b1_ref, w2_ref, b2_ref, w3_ref, b3_ref,
               out_ref, *, rows):
    feat = rows_ref[...] - cpad_ref[...]
    h = jnp.tanh(_mm(feat, w1_ref[...]) + b1_ref[...])
    h = jnp.tanh(_mm(h, w2_ref[...]) + b2_ref[...])
    h = jnp.tanh(_mm(h, w3_ref[...]) + b3_ref[...])
    m = rows // KG
    acc = h[0:m, :]
    for t in range(1, KG):
        acc = jnp.maximum(acc, h[t * m:(t + 1) * m, :])
    out_ref[...] = acc


def _w3(ux, uy, uz, kx, ky, kz, k_valid):
    D = jnp.sqrt((ux - kx) ** 2 + (uy - ky) ** 2 + (uz - kz) ** 2 + EPS)
    kpad = D.shape[1]
    col_iota = jax.lax.broadcasted_iota(jnp.int32, D.shape, 1)
    if k_valid < kpad:
        D = jnp.where(col_iota < k_valid, D, jnp.float32(BIG))
    vals, sels = [], []
    for _ in range(3):
        vmin = jnp.min(D, axis=1, keepdims=True)
        sel = jnp.min(jnp.where(D == vmin, col_iota, kpad),
                      axis=1, keepdims=True)
        vals.append(vmin)
        sels.append(sel)
        D = jnp.where(col_iota == sel, jnp.float32(BIG), D)
    ws = [1.0 / jnp.maximum(v, 1e-10) for v in vals]
    wsum = ws[0] + ws[1] + ws[2]
    Wmat = jnp.zeros_like(D)
    for t in range(3):
        Wmat = Wmat + jnp.where(col_iota == sels[t], ws[t] / wsum, 0.0)
    return Wmat


def _interp_kernel(ux_ref, uy_ref, uz_ref, kx_ref, ky_ref, kz_ref,
                   cur_ref, pospad_ref, w1a_ref, w1b_ref, b1_ref,
                   w2_ref, b2_ref, out_ref, *, k_valid):
    Wmat = _w3(ux_ref[...], uy_ref[...], uz_ref[...],
               kx_ref[...], ky_ref[...], kz_ref[...], k_valid)
    interp = _mm(Wmat, cur_ref[...])
    h = jnp.tanh(_mm(pospad_ref[...], w1a_ref[...])
                 + _mm(interp, w1b_ref[...]) + b1_ref[...])
    out_ref[...] = jnp.tanh(_mm(h, w2_ref[...]) + b2_ref[...])


def _tail_kernel(grp2_ref, cpad2_ref,
                 v1_ref, vb1_ref, v2_ref, vb2_ref, v3_ref, vb3_ref,
                 cps2p_ref, crow_ref,
                 g1a_ref, g1b_ref, bg1_ref, g2_ref, bg2_ref, g3_ref, bg3_ref,
                 d1a_ref, d1b_ref, bd1_ref, d2_ref, bd2_ref,
                 p2x_ref, p2y_ref, p2z_ref, k2x_ref, k2y_ref, k2z_ref,
                 cps1p_ref, e1a_ref, e1b_ref, be1_ref, e2_ref, be2_ref,
                 p1x_ref, p1y_ref, p1z_ref, k1x_ref, k1y_ref, k1z_ref,
                 cps0p_ref, f1a_ref, f1b_ref, bf1_ref, f2_ref, bf2_ref,
                 out_ref):
    feat2 = grp2_ref[...] - cpad2_ref[...]
    h = jnp.tanh(_mm(feat2, v1_ref[...]) + vb1_ref[...])
    h = jnp.tanh(_mm(h, v2_ref[...]) + vb2_ref[...])
    h = jnp.tanh(_mm(h, v3_ref[...]) + vb3_ref[...])
    pooled2 = h[0:16, :]
    for t in range(1, KG):
        pooled2 = jnp.maximum(pooled2, h[t * 16:(t + 1) * 16, :])

    cps2p = cps2p_ref[...]
    hg = jnp.tanh(_mm(cps2p - crow_ref[...], g1a_ref[...])
                  + _mm(pooled2, g1b_ref[...]) + bg1_ref[...])
    hg = jnp.tanh(_mm(hg, g2_ref[...]) + bg2_ref[...])
    hg = jnp.tanh(_mm(hg, g3_ref[...]) + bg3_ref[...])
    pg = jnp.max(hg, axis=0, keepdims=True)
    cur3 = jnp.tanh(_mm(cps2p, d1a_ref[...]) + _mm(pg, d1b_ref[...])
                    + bd1_ref[...])
    cur3 = jnp.tanh(_mm(cur3, d2_ref[...]) + bd2_ref[...])

    Wm2 = _w3(p2x_ref[...], p2y_ref[...], p2z_ref[...],
              k2x_ref[...], k2y_ref[...], k2z_ref[...], 16)
    interp2 = _mm(Wm2[:, 0:16], cur3)
    cur2 = jnp.tanh(_mm(cps1p_ref[...], e1a_ref[...])
                    + _mm(interp2, e1b_ref[...]) + be1_ref[...])
    cur2 = jnp.tanh(_mm(cur2, e2_ref[...]) + be2_ref[...])

    Wm1 = _w3(p1x_ref[...], p1y_ref[...], p1z_ref[...],
              k1x_ref[...], k1y_ref[...], k1z_ref[...], 64)
    interp1 = _mm(Wm1[:, 0:64], cur2)
    h1 = jnp.tanh(_mm(cps0p_ref[...], f1a_ref[...])
                  + _mm(interp1, f1b_ref[...]) + bf1_ref[...])
    out_ref[...] = jnp.tanh(_mm(h1, f2_ref[...]) + bf2_ref[...])


def _pad_cols(a, width):
    return jnp.concatenate(
        [a, jnp.zeros((a.shape[0], width - a.shape[1]), a.dtype)], axis=1)


def _pad_rows(a, height):
    return jnp.concatenate(
        [a, jnp.zeros((height - a.shape[0], a.shape[1]), a.dtype)], axis=0)


def _f32(shape):
    return jax.ShapeDtypeStruct(shape, jnp.float32)


def _i32(shape):
    return jax.ShapeDtypeStruct(shape, jnp.int32)


def _run_sa(nbr_flat, data, cpad, layers, rows, cout):
    w1, b1 = layers[0]
    w2, b2 = layers[1]
    w3, b3 = layers[2]
    cpadw = data.shape[1]
    w1p = _pad_rows(w1, cpadw)
    grp = _sc_gather(data, nbr_flat)
    fn = functools.partial(_sa_kernel, rows=rows)
    return pl.pallas_call(
        fn,
        out_shape=_f32((rows // KG, cout)),
    )(grp, cpad,
      w1p, b1.reshape(1, -1), w2, b2.reshape(1, -1), w3, b3.reshape(1, -1))


def _run_interp(upcols, krow, cur_pad, pospad, layers, k_valid, uk, cout):
    (w1, b1), (w2, b2) = layers
    w1a = _pad_rows(w1[:3], 128)
    w1b = w1[3:]
    fn = functools.partial(_interp_kernel, k_valid=k_valid)
    return pl.pallas_call(
        fn,
        out_shape=_f32((uk, cout)),
    )(upcols[0], upcols[1], upcols[2], krow[0], krow[1], krow[2],
      cur_pad, pospad, w1a, w1b, b1.reshape(1, -1), w2, b2.reshape(1, -1))


def kernel(inputs, params):
    pos = inputs[:, :3]
    xs = inputs[:, 0].reshape(32, 128)
    ys = inputs[:, 1].reshape(32, 128)
    zs = inputs[:, 2].reshape(32, 128)

    a_out = pl.pallas_call(
        _fps_kernel,
        in_specs=[pl.BlockSpec(memory_space=pltpu.VMEM)] * 3
        + [pl.BlockSpec(memory_space=pltpu.SMEM)] * 3,
        out_shape=[_i32((4, 128)), _f32((4, 128)), _f32((4, 128)),
                   _f32((4, 128)),
                   _i32((1, 128)), _f32((1, 128)), _f32((1, 128)),
                   _f32((1, 128)),
                   _i32((1, 128)), _f32((1, 128)), _f32((1, 128)),
                   _f32((1, 128)),
                   _f32((1, 128))],
        scratch_shapes=[pltpu.SMEM((512,), jnp.int32)],
    )(xs, ys, zs, inputs[:, 0], inputs[:, 1], inputs[:, 2])
    (idx0, c0x, c0y, c0z, idx1, c1x, c1y, c1z,
     idx2, c2x, c2y, c2z, scal) = a_out

    conv = params["conv"]
    conv_re = params["conv_re"]

    pcols0 = [pos[:, 0:1], pos[:, 1:2], pos[:, 2:3]]
    crow0 = [c0x.reshape(1, 512), c0y.reshape(1, 512), c0z.reshape(1, 512)]
    cps0 = jnp.stack([c0x.reshape(512), c0y.reshape(512),
                      c0z.reshape(512)], axis=1)
    pcols1 = [cps0[:, 0:1], cps0[:, 1:2], cps0[:, 2:3]]
    crow1 = [c1x, c1y, c1z]
    cps1 = jnp.stack([c1x[0, :64], c1y[0, :64], c1z[0, :64]], axis=1)
    pcols2 = [cps1[:, 0:1], cps1[:, 1:2], cps1[:, 2:3]]
    crow2 = [c2x, c2y, c2z]
    cps2 = jnp.stack([c2x[0, :16], c2y[0, :16], c2z[0, :16]], axis=1)

    nbrT0, nbrT1, nbrT2 = pl.pallas_call(
        _topk3_kernel,
        in_specs=[pl.BlockSpec(memory_space=pltpu.VMEM)] * 21
        + [pl.BlockSpec(memory_space=pltpu.SMEM)],
        out_shape=[_i32((KG, 512)), _i32((KG, 128)), _i32((KG, 128))],
    )(pcols0[0], pcols0[1], pcols0[2], crow0[0], crow0[1], crow0[2],
      idx0.reshape(1, 512),
      pcols1[0], pcols1[1], pcols1[2], crow1[0], crow1[1], crow1[2], idx1,
      pcols2[0], pcols2[1], pcols2[2], crow2[0], crow2[1], crow2[2], idx2,
      scal[0, :3])

    cpad0 = jnp.tile(_pad_cols(cps0, 128), (KG, 1))
    pooled0 = _run_sa(nbrT0.reshape(-1), _pad_cols(inputs, 128), cpad0,
                      conv[0], KG * 512, 128)

    data1 = _pad_cols(jnp.concatenate([cps0, pooled0], axis=1), 256)
    cpad1 = jnp.tile(_pad_cols(cps1, 256), (KG, 1))
    pooled1 = _run_sa(nbrT1[:, :64].reshape(-1), data1, cpad1,
                      conv[1], KG * 64, 256)

    data2 = _pad_cols(jnp.concatenate([cps1, pooled1], axis=1), 384)
    cpad2 = jnp.tile(_pad_cols(cps2, 384), (KG, 1))
    grp2 = _sc_gather(data2, nbrT2[:, :16].reshape(-1))
    (v1, vb1), (v2, vb2), (v3, vb3) = conv[2]
    (g1, bg1), (g2, bg2), (g3, bg3) = conv[3]
    (rd1, rb1), (rd2, rb2) = conv_re[0]
    (e1, be1), (e2, be2) = conv_re[1]
    (f1, bf1), (f2, bf2) = conv_re[2]
    cur1 = pl.pallas_call(
        _tail_kernel,
        out_shape=_f32((512, 128)),
    )(grp2, cpad2,
      _pad_rows(v1, 384), vb1.reshape(1, -1), v2, vb2.reshape(1, -1),
      v3, vb3.reshape(1, -1),
      _pad_cols(cps2, 128), _pad_cols(scal[0:1, 3:6], 128),
      _pad_rows(g1[:3], 128), g1[3:], bg1.reshape(1, -1),
      g2, bg2.reshape(1, -1), g3, bg3.reshape(1, -1),
      _pad_rows(rd1[:3], 128), rd1[3:], rb1.reshape(1, -1),
      rd2, rb2.reshape(1, -1),
      pcols2[0], pcols2[1], pcols2[2], crow2[0], crow2[1], crow2[2],
      _pad_cols(cps1, 128), _pad_rows(e1[:3], 128), e1[3:],
      be1.reshape(1, -1), e2, be2.reshape(1, -1),
      pcols1[0], pcols1[1], pcols1[2], crow1[0], crow1[1], crow1[2],
      _pad_cols(cps0, 128), _pad_rows(f1[:3], 128), f1[3:],
      bf1.reshape(1, -1), f2, bf2.reshape(1, -1))

    out = _run_interp(pcols0, crow0, cur1,
                      _pad_cols(pos, 128), conv_re[3], 512, 4096, 128)
    return out
